# Initial kernel scaffold; baseline (speedup 1.0000x reference)
import jax
import jax.numpy as jnp
from jax.experimental import pallas as pl


def kernel(xyz1, xyz2):
    raise NotImplementedError("write your pallas kernel here")


# speedup vs baseline: 1.0646x; 1.0646x over previous
import functools

import jax
import jax.numpy as jnp
from jax.experimental import pallas as pl
from jax.experimental.pallas import tpu as pltpu


def _chamfer_body(x1_ref, x2_ref, out_ref, d2min_ref, *, ni_blocks, inv_n):
    b = pl.program_id(0)
    i = pl.program_id(1)

    x1 = x1_ref[0]
    x2 = x2_ref[0]

    inner = jax.lax.dot_general(
        x1, x2, (((0,), (0,)), ((), ())), preferred_element_type=jnp.float32
    )
    sq1 = jnp.sum(x1 * x1, axis=0, keepdims=True)
    sq2 = jnp.sum(x2 * x2, axis=0, keepdims=True)

    e = (sq1.T - 2.0 * inner) + sq2

    row_min = jnp.min(e, axis=1, keepdims=True)
    s1 = jnp.sum(row_min)

    col_min = jnp.min(e, axis=0, keepdims=True)

    @pl.when(i == 0)
    def _init():
        d2min_ref[...] = col_min

    @pl.when(i > 0)
    def _acc():
        d2min_ref[...] = jnp.minimum(d2min_ref[...], col_min)

    @pl.when(jnp.logical_and(b == 0, i == 0))
    def _zero():
        out_ref[0, 0] = 0.0

    out_ref[0, 0] += s1 * inv_n

    @pl.when(i == ni_blocks - 1)
    def _flush():
        out_ref[0, 0] += jnp.sum(d2min_ref[...]) * inv_n


def kernel(xyz1, xyz2):
    B, N1, _ = xyz1.shape
    _, N2, _ = xyz2.shape
    BI = 512
    ni_blocks = N1 // BI

    x1t = jnp.transpose(xyz1, (0, 2, 1))
    x2t = jnp.transpose(xyz2, (0, 2, 1))

    body = functools.partial(
        _chamfer_body, ni_blocks=ni_blocks, inv_n=1.0 / float(B * N1)
    )

    out = pl.pallas_call(
        body,
        grid=(B, ni_blocks),
        in_specs=[
            pl.BlockSpec((1, 3, BI), lambda b, i: (b, 0, i)),
            pl.BlockSpec((1, 3, N2), lambda b, i: (b, 0, 0)),
        ],
        out_specs=pl.BlockSpec(
            (1, 1), lambda b, i: (0, 0), memory_space=pltpu.SMEM
        ),
        out_shape=jax.ShapeDtypeStruct((1, 1), jnp.float32),
        scratch_shapes=[pltpu.VMEM((1, N2), jnp.float32)],
    )(x1t, x2t)
    return out[0, 0]


# speedup vs baseline: 1.3967x; 1.3118x over previous
import functools

import jax
import jax.numpy as jnp
from jax.experimental import pallas as pl
from jax.experimental.pallas import tpu as pltpu


def _chamfer_body(x1_ref, x2_ref, out_ref, d2min_ref, *, ni_blocks, inv_n):
    b = pl.program_id(0)
    i = pl.program_id(1)

    x1 = x1_ref[0]
    x2 = x2_ref[0]

    bi = x1.shape[1]
    n2 = x2.shape[1]
    sq1 = jnp.sum(x1 * x1, axis=0, keepdims=True)
    sq2 = jnp.sum(x2 * x2, axis=0, keepdims=True)
    aug1 = jnp.concatenate(
        [x1 * -2.0, sq1, jnp.ones((1, bi), jnp.float32)], axis=0
    )
    aug2 = jnp.concatenate(
        [x2, jnp.ones((1, n2), jnp.float32), sq2], axis=0
    )
    e = jax.lax.dot_general(
        aug1, aug2, (((0,), (0,)), ((), ())), preferred_element_type=jnp.float32
    )

    row_min = jnp.min(e, axis=1, keepdims=True)
    s1 = jnp.sum(row_min)

    col_min = jnp.min(e, axis=0, keepdims=True)

    @pl.when(i == 0)
    def _init():
        d2min_ref[...] = col_min

    @pl.when(i > 0)
    def _acc():
        d2min_ref[...] = jnp.minimum(d2min_ref[...], col_min)

    @pl.when(jnp.logical_and(b == 0, i == 0))
    def _zero():
        out_ref[0, 0] = 0.0

    out_ref[0, 0] += s1 * inv_n

    @pl.when(i == ni_blocks - 1)
    def _flush():
        out_ref[0, 0] += jnp.sum(d2min_ref[...]) * inv_n


def kernel(xyz1, xyz2):
    B, N1, _ = xyz1.shape
    _, N2, _ = xyz2.shape
    BI = 512
    ni_blocks = N1 // BI

    x1t = jnp.transpose(xyz1, (0, 2, 1))
    x2t = jnp.transpose(xyz2, (0, 2, 1))

    body = functools.partial(
        _chamfer_body, ni_blocks=ni_blocks, inv_n=1.0 / float(B * N1)
    )

    out = pl.pallas_call(
        body,
        grid=(B, ni_blocks),
        in_specs=[
            pl.BlockSpec((1, 3, BI), lambda b, i: (b, 0, i)),
            pl.BlockSpec((1, 3, N2), lambda b, i: (b, 0, 0)),
        ],
        out_specs=pl.BlockSpec(
            (1, 1), lambda b, i: (0, 0), memory_space=pltpu.SMEM
        ),
        out_shape=jax.ShapeDtypeStruct((1, 1), jnp.float32),
        scratch_shapes=[pltpu.VMEM((1, N2), jnp.float32)],
    )(x1t, x2t)
    return out[0, 0]
